# baseline (speedup 1.0000x reference)
import functools

import jax
import jax.numpy as jnp
from jax import lax
from jax.experimental import pallas as pl
from jax.experimental.pallas import tpu as pltpu
from jax.experimental.pallas import tpu_sc as plsc

_K = 8192
_D = 256
_TAU = 0.2
_C2 = 1.4426950408889634 / _TAU
_EPS = 1e-12
_BR = 512
_N = 32 * 576


def _main_body(x_ref, w_ref, idx_ref, loss_ref, wn_ref, wc_ref):
    i = pl.program_id(0)

    @pl.when(i == 0)
    def _init():
        w = w_ref[...]
        n = jnp.sqrt(jnp.sum(w * w, axis=1, keepdims=True))
        wnf = w / jnp.maximum(n, _EPS)
        wn_ref[...] = wnf.astype(jnp.bfloat16)
        wc_ref[...] = (wnf * _C2).astype(jnp.bfloat16)
        loss_ref[...] = jnp.zeros_like(loss_ref)

    x = x_ref[...]
    n = jnp.sqrt(jnp.sum(x * x, axis=1, keepdims=True))
    xn = (x / jnp.maximum(n, _EPS)).astype(jnp.bfloat16)
    cos = lax.dot_general(
        xn, wn_ref[...], (((1,), (1,)), ((), ())),
        preferred_element_type=jnp.float32)
    m = jnp.max(cos, axis=1, keepdims=True)
    iota = lax.broadcasted_iota(jnp.int32, cos.shape, 1).astype(jnp.float32)
    idxf = jnp.min(jnp.where(cos == m, iota, float(_K)), axis=1,
                   keepdims=True)
    idx_ref[...] = idxf.astype(jnp.int32)
    cos2 = lax.dot_general(
        xn, wc_ref[...], (((1,), (1,)), ((), ())),
        preferred_element_type=jnp.float32)
    s = jnp.sum(jnp.exp2(cos2), axis=1, keepdims=True)
    row_loss = jnp.log(s) - m * (1.0 / _TAU)
    loss_ref[...] = loss_ref[...] + jnp.sum(row_loss).reshape(1, 1)


def _compute_indices_loss(latent, w):
    return pl.pallas_call(
        _main_body,
        grid=(_N // _BR,),
        in_specs=[
            pl.BlockSpec((_BR, _D), lambda i: (i, 0)),
            pl.BlockSpec((_K, _D), lambda i: (0, 0)),
        ],
        out_specs=[
            pl.BlockSpec((_BR, 1), lambda i: (i, 0)),
            pl.BlockSpec((1, 1), lambda i: (0, 0)),
        ],
        out_shape=[
            jax.ShapeDtypeStruct((_N, 1), jnp.int32),
            jax.ShapeDtypeStruct((1, 1), jnp.float32),
        ],
        scratch_shapes=[pltpu.VMEM((_K, _D), jnp.bfloat16),
                        pltpu.VMEM((_K, _D), jnp.bfloat16)],
    )(latent, w)


_NW = 32
_BPW = _N // _NW
_CHUNKS = 2
_BC = _BPW // _CHUNKS


def _gather_rows(table, idx):
    mesh = plsc.VectorSubcoreMesh(core_axis_name="c", subcore_axis_name="s")
    info = plsc.get_sparse_core_info()
    nc = info.num_cores

    @functools.partial(
        pl.kernel, mesh=mesh,
        out_type=jax.ShapeDtypeStruct((_N, _D), jnp.float32),
        scratch_types=[
            pltpu.VMEM((_BC,), jnp.int32),
            pltpu.VMEM((_BC, _D), jnp.float32),
            pltpu.SemaphoreType.DMA,
        ],
    )
    def k(table_hbm, idx_hbm, out_hbm, idx_v, rows_v, sem):
        wid = lax.axis_index("s") * nc + lax.axis_index("c")
        for c in range(_CHUNKS):
            base = wid * _BPW + c * _BC
            pltpu.sync_copy(idx_hbm.at[pl.ds(base, _BC)], idx_v)
            pltpu.async_copy(table_hbm.at[idx_v], rows_v, sem).wait()
            pltpu.sync_copy(rows_v, out_hbm.at[pl.ds(base, _BC)])

    return k(table, idx)


def kernel(x, W):
    latent = x.reshape(-1, _D)
    idx2, loss_sum = _compute_indices_loss(latent, W)
    idx = idx2.reshape(-1)
    x_q = _gather_rows(W, idx)
    loss = loss_sum[0, 0] / latent.shape[0]
    return (x_q.reshape(x.shape), loss, idx.reshape(x.shape[:-1]))


# speedup vs baseline: 1.1891x; 1.1891x over previous
import functools

import jax
import jax.numpy as jnp
from jax import lax
from jax.experimental import pallas as pl
from jax.experimental.pallas import tpu as pltpu
from jax.experimental.pallas import tpu_sc as plsc

_K = 8192
_D = 256
_TAU = 0.2
_C2 = 1.4426950408889634 / _TAU
_EPS = 1e-12
_BR = 1152
_N = 32 * 576


def _main_body(x_ref, w_ref, idx_ref, loss_ref, wn_ref):
    i = pl.program_id(0)

    @pl.when(i == 0)
    def _init():
        w = w_ref[...]
        n = jnp.sqrt(jnp.sum(w * w, axis=1, keepdims=True))
        wn_ref[...] = (w / jnp.maximum(n, _EPS)).astype(jnp.bfloat16)
        loss_ref[...] = jnp.zeros_like(loss_ref)

    x = x_ref[...]
    n = jnp.sqrt(jnp.sum(x * x, axis=1, keepdims=True))
    xn = (x / jnp.maximum(n, _EPS)).astype(jnp.bfloat16)
    cos = lax.dot_general(
        xn, wn_ref[...], (((1,), (1,)), ((), ())),
        preferred_element_type=jnp.float32)
    m = jnp.max(cos, axis=1, keepdims=True)
    iota = lax.broadcasted_iota(jnp.int32, cos.shape, 1).astype(jnp.float32)
    idxf = jnp.min(jnp.where(cos == m, iota, float(_K)), axis=1,
                   keepdims=True)
    idx_ref[...] = idxf.astype(jnp.int32)
    s = jnp.sum(jnp.exp2(cos * _C2), axis=1, keepdims=True)
    row_loss = jnp.log(s) - m * (1.0 / _TAU)
    loss_ref[...] = loss_ref[...] + jnp.sum(row_loss).reshape(1, 1)


def _compute_indices_loss(latent, w):
    return pl.pallas_call(
        _main_body,
        grid=(_N // _BR,),
        in_specs=[
            pl.BlockSpec((_BR, _D), lambda i: (i, 0)),
            pl.BlockSpec((_K, _D), lambda i: (0, 0)),
        ],
        out_specs=[
            pl.BlockSpec((_BR, 1), lambda i: (i, 0)),
            pl.BlockSpec((1, 1), lambda i: (0, 0)),
        ],
        out_shape=[
            jax.ShapeDtypeStruct((_N, 1), jnp.int32),
            jax.ShapeDtypeStruct((1, 1), jnp.float32),
        ],
        scratch_shapes=[pltpu.VMEM((_K, _D), jnp.bfloat16)],
    )(latent, w)


_NW = 32
_BPW = _N // _NW
_CHUNKS = 2
_BC = _BPW // _CHUNKS


def _gather_rows(table, idx):
    mesh = plsc.VectorSubcoreMesh(core_axis_name="c", subcore_axis_name="s")
    info = plsc.get_sparse_core_info()
    nc = info.num_cores

    @functools.partial(
        pl.kernel, mesh=mesh,
        out_type=jax.ShapeDtypeStruct((_N, _D), jnp.float32),
        scratch_types=[
            pltpu.VMEM((_BC,), jnp.int32),
            pltpu.VMEM((_BC, _D), jnp.float32),
            pltpu.SemaphoreType.DMA,
        ],
    )
    def k(table_hbm, idx_hbm, out_hbm, idx_v, rows_v, sem):
        wid = lax.axis_index("s") * nc + lax.axis_index("c")
        for c in range(_CHUNKS):
            base = wid * _BPW + c * _BC
            pltpu.sync_copy(idx_hbm.at[pl.ds(base, _BC)], idx_v)
            pltpu.async_copy(table_hbm.at[idx_v], rows_v, sem).wait()
            pltpu.sync_copy(rows_v, out_hbm.at[pl.ds(base, _BC)])

    return k(table, idx)


def kernel(x, W):
    latent = x.reshape(-1, _D)
    idx2, loss_sum = _compute_indices_loss(latent, W)
    idx = idx2.reshape(-1)
    x_q = _gather_rows(W, idx)
    loss = loss_sum[0, 0] / latent.shape[0]
    return (x_q.reshape(x.shape), loss, idx.reshape(x.shape[:-1]))


# speedup vs baseline: 1.1894x; 1.0002x over previous
import functools

import jax
import jax.numpy as jnp
from jax import lax
from jax.experimental import pallas as pl
from jax.experimental.pallas import tpu as pltpu
from jax.experimental.pallas import tpu_sc as plsc

_K = 8192
_D = 256
_TAU = 0.2
_C2 = 1.4426950408889634 / _TAU
_EPS = 1e-12
_BR = 1152
_N = 32 * 576


def _main_body(x_ref, w_ref, idx_ref, loss_ref, wn_ref):
    i = pl.program_id(0)

    @pl.when(i == 0)
    def _init():
        w = w_ref[...]
        n = jnp.sqrt(jnp.sum(w * w, axis=1, keepdims=True))
        wn_ref[...] = (w / jnp.maximum(n, _EPS)).astype(jnp.bfloat16)
        loss_ref[...] = jnp.zeros_like(loss_ref)

    x = x_ref[...]
    n = jnp.sqrt(jnp.sum(x * x, axis=1, keepdims=True))
    xn = (x / jnp.maximum(n, _EPS)).astype(jnp.bfloat16)
    cos = lax.dot_general(
        xn, wn_ref[...], (((1,), (1,)), ((), ())),
        preferred_element_type=jnp.float32)
    m = jnp.max(cos, axis=1, keepdims=True)
    iota = lax.broadcasted_iota(jnp.int32, cos.shape, 1).astype(jnp.float32)
    idxf = jnp.min(jnp.where(cos == m, iota, float(_K)), axis=1,
                   keepdims=True)
    idx_ref[...] = idxf.astype(jnp.int32)
    s = jnp.sum(jnp.exp2(cos * _C2), axis=1, keepdims=True)
    row_loss = jnp.log(s) - m * (1.0 / _TAU)
    loss_ref[...] = loss_ref[...] + jnp.sum(row_loss).reshape(1, 1)

    @pl.when(i == _N // _BR - 1)
    def _final():
        loss_ref[...] = loss_ref[...] * (1.0 / _N)


def _compute_indices_loss(latent, w):
    return pl.pallas_call(
        _main_body,
        grid=(_N // _BR,),
        in_specs=[
            pl.BlockSpec((_BR, _D), lambda i: (i, 0)),
            pl.BlockSpec((_K, _D), lambda i: (0, 0)),
        ],
        out_specs=[
            pl.BlockSpec((_BR, 1), lambda i: (i, 0)),
            pl.BlockSpec((1, 1), lambda i: (0, 0)),
        ],
        out_shape=[
            jax.ShapeDtypeStruct((_N, 1), jnp.int32),
            jax.ShapeDtypeStruct((1, 1), jnp.float32),
        ],
        scratch_shapes=[pltpu.VMEM((_K, _D), jnp.bfloat16)],
    )(latent, w)


_NW = 32
_BPW = _N // _NW
_CHUNKS = 3
_BC = _BPW // _CHUNKS


def _gather_rows(table, idx):
    mesh = plsc.VectorSubcoreMesh(core_axis_name="c", subcore_axis_name="s")
    info = plsc.get_sparse_core_info()
    nc = info.num_cores

    @functools.partial(
        pl.kernel, mesh=mesh,
        out_type=jax.ShapeDtypeStruct((_N, _D), jnp.float32),
        scratch_types=[
            pltpu.VMEM((_BPW,), jnp.int32),
            pltpu.VMEM((_BC, _D), jnp.float32),
            pltpu.VMEM((_BC, _D), jnp.float32),
            pltpu.SemaphoreType.DMA,
            pltpu.SemaphoreType.DMA,
            pltpu.SemaphoreType.DMA,
        ],
    )
    def k(table_hbm, idx_hbm, out_hbm, idx_v, rows0, rows1, sem_g, sw0, sw1):
        wid = lax.axis_index("s") * nc + lax.axis_index("c")
        base = wid * _BPW
        pltpu.sync_copy(idx_hbm.at[pl.ds(base, _BPW)], idx_v)
        bufs = (rows0, rows1)
        sems = (sw0, sw1)
        writes = [None, None]
        for c in range(_CHUNKS):
            b = c % 2
            if writes[b] is not None:
                writes[b].wait()
            pltpu.async_copy(
                table_hbm.at[idx_v.at[pl.ds(c * _BC, _BC)]], bufs[b],
                sem_g).wait()
            writes[b] = pltpu.async_copy(
                bufs[b], out_hbm.at[pl.ds(base + c * _BC, _BC)], sems[b])
        for w in writes:
            if w is not None:
                w.wait()

    return k(table, idx)


def kernel(x, W):
    latent = x.reshape(-1, _D)
    idx2, loss_sum = _compute_indices_loss(latent, W)
    idx = idx2.reshape(-1)
    x_q = _gather_rows(W, idx)
    return (x_q.reshape(x.shape), loss_sum[0, 0], idx.reshape(x.shape[:-1]))
